# baseline (speedup 1.0000x reference)
import functools

import jax
import jax.numpy as jnp
from jax import lax
from jax.experimental import pallas as pl
from jax.experimental.pallas import tpu as pltpu
from jax.experimental.pallas import tpu_sc as plsc

N = 10000
NF = 128
D1 = 64
D2 = 32
NC, NS = 2, 16
CH = 128
NCH = 82
EPW = NCH * CH
EPAD = NC * NS * EPW
NP = 10240
RPS = NP // NS
RB = 1000


def _sc_mesh():
    return plsc.VectorSubcoreMesh(
        core_axis_name="c", subcore_axis_name="s", num_cores=NC, num_subcores=NS
    )


@functools.lru_cache(maxsize=None)
def _make_agg(D):
    @functools.partial(
        pl.kernel,
        out_type=jax.ShapeDtypeStruct((NC, NP, D), jnp.float32),
        mesh=_sc_mesh(),
        scratch_types=[
            pltpu.VMEM((NCH, CH), jnp.int32),
            pltpu.VMEM((NCH, CH), jnp.int32),
            pltpu.VMEM((CH, D), jnp.bfloat16),
            pltpu.VMEM((CH, D), jnp.bfloat16),
            pltpu.VMEM((CH, D), jnp.float32),
            pltpu.VMEM_SHARED((NP, D), jnp.float32),
            pltpu.SemaphoreType.DMA,
            pltpu.SemaphoreType.DMA,
        ],
        compiler_params=pltpu.CompilerParams(use_tc_tiling_on_sc=False),
    )
    def agg(m_hbm, src_hbm, dst_hbm, out_hbm, src_v, dst_v, rb0, rb1,
            rows_f, acc_sh, sem0, sem1):
        c = lax.axis_index("c")
        s = lax.axis_index("s")
        pltpu.sync_copy(src_hbm.at[c, s], src_v)
        pltpu.sync_copy(dst_hbm.at[c, s], dst_v)

        zero16 = jnp.zeros((16,), jnp.float32)

        def zrow(i, carry):
            for j in range(D // 16):
                rows_f[i, pl.ds(j * 16, 16)] = zero16
            return carry

        lax.fori_loop(0, CH, zrow, 0)
        base = s * RPS
        for k in range(RPS // CH):
            pltpu.sync_copy(rows_f, acc_sh.at[pl.ds(base + k * CH, CH)])
        plsc.subcore_barrier()

        def convert(rb):
            def crow(i, carry):
                for k in range(D // 32):
                    v = rb[i, pl.ds(32 * k, 32)].reshape(2, 16)
                    f = v.astype(jnp.float32)
                    rows_f[i, pl.ds(32 * k, 16)] = f[0]
                    rows_f[i, pl.ds(32 * k + 16, 16)] = f[1]
                return carry

            lax.fori_loop(0, CH, crow, 0)

        NT = NCH // 2
        pltpu.async_copy(m_hbm.at[src_v.at[0]], rb0, sem0)

        def body(t, carry):
            j0 = 2 * t
            j1 = j0 + 1
            pltpu.async_copy(m_hbm.at[src_v.at[j1]], rb1, sem1)
            pltpu.make_async_copy(m_hbm.at[src_v.at[j0]], rb0, sem0).wait()
            convert(rb0)
            pltpu.sync_copy(rows_f, acc_sh.at[dst_v.at[j0]], add=True)

            @pl.when(t < NT - 1)
            def _():
                pltpu.async_copy(m_hbm.at[src_v.at[j0 + 2]], rb0, sem0)

            pltpu.make_async_copy(m_hbm.at[src_v.at[j1]], rb1, sem1).wait()
            convert(rb1)
            pltpu.sync_copy(rows_f, acc_sh.at[dst_v.at[j1]], add=True)
            return carry

        lax.fori_loop(0, NT, body, 0)
        plsc.subcore_barrier()
        pltpu.sync_copy(acc_sh.at[pl.ds(base, RPS)], out_hbm.at[c, pl.ds(base, RPS)])

    return agg


@functools.lru_cache(maxsize=None)
def _make_deg():
    @functools.partial(
        pl.kernel,
        out_type=jax.ShapeDtypeStruct((NC, NP, 16), jnp.float32),
        mesh=_sc_mesh(),
        scratch_types=[
            pltpu.VMEM((NCH, CH), jnp.int32),
            pltpu.VMEM((CH, 16), jnp.float32),
            pltpu.VMEM_SHARED((NP, 16), jnp.float32),
        ],
        compiler_params=pltpu.CompilerParams(use_tc_tiling_on_sc=False),
    )
    def deg(dst_hbm, out_hbm, dst_v, rows_v, acc_sh):
        c = lax.axis_index("c")
        s = lax.axis_index("s")
        pltpu.sync_copy(dst_hbm.at[c, s], dst_v)

        zero16 = jnp.zeros((16,), jnp.float32)

        def zrow(i, carry):
            rows_v[i, pl.ds(0, 16)] = zero16
            return carry

        lax.fori_loop(0, CH, zrow, 0)
        base = s * RPS
        for k in range(RPS // CH):
            pltpu.sync_copy(rows_v, acc_sh.at[pl.ds(base + k * CH, CH)])
        plsc.subcore_barrier()

        one16 = jnp.ones((16,), jnp.float32)

        def orow(i, carry):
            rows_v[i, pl.ds(0, 16)] = one16
            return carry

        lax.fori_loop(0, CH, orow, 0)

        def body(j, carry):
            pltpu.sync_copy(rows_v, acc_sh.at[dst_v.at[j]], add=True)
            return carry

        lax.fori_loop(0, NCH, body, 0)
        plsc.subcore_barrier()
        pltpu.sync_copy(acc_sh.at[pl.ds(base, RPS)], out_hbm.at[c, pl.ds(base, RPS)])

    return deg


def _tc_prep(degP, x, W1):
    def body(degp, x_ref, w_ref, dinv_ref, m_ref):
        d = degp[0, :, 0:1] + degp[1, :, 0:1]
        dinv = lax.rsqrt(d)
        dinv_ref[...] = dinv
        m_ref[...] = ((x_ref[...] @ w_ref[...]) * dinv).astype(jnp.bfloat16)

    return pl.pallas_call(
        body,
        grid=(N // RB,),
        in_specs=[
            pl.BlockSpec((NC, RB, 16), lambda i: (0, i, 0)),
            pl.BlockSpec((RB, NF), lambda i: (i, 0)),
            pl.BlockSpec((NF, D1), lambda i: (0, 0)),
        ],
        out_specs=[
            pl.BlockSpec((RB, 1), lambda i: (i, 0)),
            pl.BlockSpec((RB, D1), lambda i: (i, 0)),
        ],
        out_shape=[
            jax.ShapeDtypeStruct((N, 1), jnp.float32),
            jax.ShapeDtypeStruct((N, D1), jnp.bfloat16),
        ],
    )(degP, x, W1)


def _tc_mid(P, b, dinv, W):
    D = P.shape[2]
    Dn = W.shape[1]

    def body(p_ref, b_ref, dinv_ref, w_ref, m_ref):
        dinv = dinv_ref[...]
        h = jnp.maximum(dinv * (p_ref[0] + p_ref[1]) + b_ref[...], 0.0)
        m_ref[...] = ((h @ w_ref[...]) * dinv).astype(jnp.bfloat16)

    return pl.pallas_call(
        body,
        grid=(N // RB,),
        in_specs=[
            pl.BlockSpec((NC, RB, D), lambda i: (0, i, 0)),
            pl.BlockSpec((1, D), lambda i: (0, 0)),
            pl.BlockSpec((RB, 1), lambda i: (i, 0)),
            pl.BlockSpec((D, Dn), lambda i: (0, 0)),
        ],
        out_specs=pl.BlockSpec((RB, Dn), lambda i: (i, 0)),
        out_shape=jax.ShapeDtypeStruct((N, Dn), jnp.bfloat16),
    )(P, b, dinv, W)


def _tc_mid_id(P, b, dinv):
    D = P.shape[2]

    def body(p_ref, b_ref, dinv_ref, m_ref):
        dinv = dinv_ref[...]
        h = jnp.maximum(dinv * (p_ref[0] + p_ref[1]) + b_ref[...], 0.0)
        m_ref[...] = (h * dinv).astype(jnp.bfloat16)

    return pl.pallas_call(
        body,
        grid=(N // RB,),
        in_specs=[
            pl.BlockSpec((NC, RB, D), lambda i: (0, i, 0)),
            pl.BlockSpec((1, D), lambda i: (0, 0)),
            pl.BlockSpec((RB, 1), lambda i: (i, 0)),
        ],
        out_specs=pl.BlockSpec((RB, D), lambda i: (i, 0)),
        out_shape=jax.ShapeDtypeStruct((N, D), jnp.bfloat16),
    )(P, b, dinv)


def _tc_post(P, W3, b3, dinv, W1n):
    D = P.shape[2]

    def body(p_ref, w3_ref, b3_ref, dinv_ref, *rest):
        dinv = dinv_ref[...]
        g = dinv * (p_ref[0] + p_ref[1])
        t = g @ w3_ref[...] + b3_ref[...]
        h = jax.nn.softmax(t, axis=-1)
        if W1n is None:
            rest[-1][...] = h
        else:
            w1_ref = rest[0]
            rest[-1][...] = ((h @ w1_ref[...]) * dinv).astype(jnp.bfloat16)

    in_specs = [
        pl.BlockSpec((NC, RB, D), lambda i: (0, i, 0)),
        pl.BlockSpec((D, NF), lambda i: (0, 0)),
        pl.BlockSpec((1, NF), lambda i: (0, 0)),
        pl.BlockSpec((RB, 1), lambda i: (i, 0)),
    ]
    args = [P, W3, b3, dinv]
    if W1n is None:
        out_dim = NF
    else:
        out_dim = D1
        in_specs.append(pl.BlockSpec((NF, D1), lambda i: (0, 0)))
        args.append(W1n)
    return pl.pallas_call(
        body,
        grid=(N // RB,),
        in_specs=in_specs,
        out_specs=pl.BlockSpec((RB, out_dim), lambda i: (i, 0)),
        out_shape=jax.ShapeDtypeStruct(
            (N, out_dim), jnp.float32 if W1n is None else jnp.bfloat16),
    )(*args)


def kernel(x, edge_index,
           W11, b11, W21, b21, W31, b31,
           W12, b12, W22, b22, W32, b32,
           W13, b13, W23, b23, W33, b33,
           W14, b14, W24, b24, W34, b34,
           W15, b15, W25, b25, W35, b35):
    params = {
        1: (W11, b11, W21, b21, W31, b31),
        2: (W12, b12, W22, b22, W32, b32),
        3: (W13, b13, W23, b23, W33, b33),
        4: (W14, b14, W24, b24, W34, b34),
        5: (W15, b15, W25, b25, W35, b35),
    }
    loop = jnp.arange(N, dtype=jnp.int32)
    src = jnp.concatenate([edge_index[0], loop])
    dst = jnp.concatenate([edge_index[1], loop])
    pad = EPAD - src.shape[0]
    src = jnp.concatenate([src, jnp.zeros((pad,), jnp.int32)])
    dst = jnp.concatenate([dst, jnp.full((pad,), N, jnp.int32)])
    srcA = src.reshape(NC, NS, NCH, CH)
    dstA = dst.reshape(NC, NS, NCH, CH)

    degP = _make_deg()(dstA)
    dinv, M = _tc_prep(degP, x, W11)

    agg64 = _make_agg(D1)
    agg32 = _make_agg(D2)
    h = None
    for blk in range(1, 6):
        w1, b1, w2, b2, w3, b3 = params[blk]
        P = agg64(M, srcA, dstA)
        M = _tc_mid(P, b1.reshape(1, -1), dinv, w2)
        P = agg32(M, srcA, dstA)
        M = _tc_mid_id(P, b2.reshape(1, -1), dinv)
        P = agg32(M, srcA, dstA)
        if blk < 5:
            M = _tc_post(P, w3, b3.reshape(1, -1), dinv, params[blk + 1][0])
        else:
            h = _tc_post(P, w3, b3.reshape(1, -1), dinv, None)
    return h


# speedup vs baseline: 1.1784x; 1.1784x over previous
import functools

import jax
import jax.numpy as jnp
from jax import lax
from jax.experimental import pallas as pl
from jax.experimental.pallas import tpu as pltpu
from jax.experimental.pallas import tpu_sc as plsc

N = 10000
NF = 128
D1 = 64
D2 = 32
NC, NS = 2, 16
CH = 128
NCH = 82
EPW = NCH * CH
EPAD = NC * NS * EPW
NP = 10240
RPS = NP // NS
RB = 1000


def _sc_mesh():
    return plsc.VectorSubcoreMesh(
        core_axis_name="c", subcore_axis_name="s", num_cores=NC, num_subcores=NS
    )


@functools.lru_cache(maxsize=None)
def _make_agg(D):
    @functools.partial(
        pl.kernel,
        out_type=jax.ShapeDtypeStruct((NC, NP, D), jnp.bfloat16),
        mesh=_sc_mesh(),
        scratch_types=[
            pltpu.VMEM((NCH, CH), jnp.int32),
            pltpu.VMEM((NCH, CH), jnp.int32),
            pltpu.VMEM((CH, D), jnp.bfloat16),
            pltpu.VMEM_SHARED((NP, D), jnp.bfloat16),
            pltpu.SemaphoreType.DMA,
        ],
        compiler_params=pltpu.CompilerParams(use_tc_tiling_on_sc=False),
    )
    def agg(m_hbm, src_hbm, dst_hbm, out_hbm, src_v, dst_v, rows_v, acc_sh, sem):
        c = lax.axis_index("c")
        s = lax.axis_index("s")
        pltpu.sync_copy(src_hbm.at[c, s], src_v)
        pltpu.sync_copy(dst_hbm.at[c, s], dst_v)

        zero32 = jnp.zeros((32,), jnp.bfloat16)

        def zrow(i, carry):
            for j in range(D // 32):
                rows_v[i, pl.ds(j * 32, 32)] = zero32
            return carry

        lax.fori_loop(0, CH, zrow, 0)
        base = s * RPS
        for k in range(RPS // CH):
            pltpu.sync_copy(rows_v, acc_sh.at[pl.ds(base + k * CH, CH)])
        plsc.subcore_barrier()

        def body(j, carry):
            pltpu.async_copy(m_hbm.at[src_v.at[j]], rows_v, sem).wait()
            pltpu.sync_copy(rows_v, acc_sh.at[dst_v.at[j]], add=True)
            return carry

        lax.fori_loop(0, NCH, body, 0)
        plsc.subcore_barrier()
        pltpu.sync_copy(acc_sh.at[pl.ds(base, RPS)], out_hbm.at[c, pl.ds(base, RPS)])

    return agg


@functools.lru_cache(maxsize=None)
def _make_deg():
    @functools.partial(
        pl.kernel,
        out_type=jax.ShapeDtypeStruct((NC, NP, 16), jnp.float32),
        mesh=_sc_mesh(),
        scratch_types=[
            pltpu.VMEM((NCH, CH), jnp.int32),
            pltpu.VMEM((CH, 16), jnp.float32),
            pltpu.VMEM_SHARED((NP, 16), jnp.float32),
        ],
        compiler_params=pltpu.CompilerParams(use_tc_tiling_on_sc=False),
    )
    def deg(dst_hbm, out_hbm, dst_v, rows_v, acc_sh):
        c = lax.axis_index("c")
        s = lax.axis_index("s")
        pltpu.sync_copy(dst_hbm.at[c, s], dst_v)

        zero16 = jnp.zeros((16,), jnp.float32)

        def zrow(i, carry):
            rows_v[i, pl.ds(0, 16)] = zero16
            return carry

        lax.fori_loop(0, CH, zrow, 0)
        base = s * RPS
        for k in range(RPS // CH):
            pltpu.sync_copy(rows_v, acc_sh.at[pl.ds(base + k * CH, CH)])
        plsc.subcore_barrier()

        one16 = jnp.ones((16,), jnp.float32)

        def orow(i, carry):
            rows_v[i, pl.ds(0, 16)] = one16
            return carry

        lax.fori_loop(0, CH, orow, 0)

        def body(j, carry):
            pltpu.sync_copy(rows_v, acc_sh.at[dst_v.at[j]], add=True)
            return carry

        lax.fori_loop(0, NCH, body, 0)
        plsc.subcore_barrier()
        pltpu.sync_copy(acc_sh.at[pl.ds(base, RPS)], out_hbm.at[c, pl.ds(base, RPS)])

    return deg


def _tc_prep(degP, x, W1):
    def body(degp, x_ref, w_ref, dinv_ref, m_ref):
        d = degp[0, :, 0:1] + degp[1, :, 0:1]
        dinv = lax.rsqrt(d)
        dinv_ref[...] = dinv
        m_ref[...] = ((x_ref[...] @ w_ref[...]) * dinv).astype(jnp.bfloat16)

    return pl.pallas_call(
        body,
        grid=(N // RB,),
        in_specs=[
            pl.BlockSpec((NC, RB, 16), lambda i: (0, i, 0)),
            pl.BlockSpec((RB, NF), lambda i: (i, 0)),
            pl.BlockSpec((NF, D1), lambda i: (0, 0)),
        ],
        out_specs=[
            pl.BlockSpec((RB, 1), lambda i: (i, 0)),
            pl.BlockSpec((RB, D1), lambda i: (i, 0)),
        ],
        out_shape=[
            jax.ShapeDtypeStruct((N, 1), jnp.float32),
            jax.ShapeDtypeStruct((N, D1), jnp.bfloat16),
        ],
    )(degP, x, W1)


def _tc_mid(P, b, dinv, W):
    D = P.shape[2]
    Dn = W.shape[1]

    def body(p_ref, b_ref, dinv_ref, w_ref, m_ref):
        dinv = dinv_ref[...]
        h = jnp.maximum(dinv * (p_ref[0].astype(jnp.float32) + p_ref[1].astype(jnp.float32)) + b_ref[...], 0.0)
        m_ref[...] = ((h @ w_ref[...]) * dinv).astype(jnp.bfloat16)

    return pl.pallas_call(
        body,
        grid=(N // RB,),
        in_specs=[
            pl.BlockSpec((NC, RB, D), lambda i: (0, i, 0)),
            pl.BlockSpec((1, D), lambda i: (0, 0)),
            pl.BlockSpec((RB, 1), lambda i: (i, 0)),
            pl.BlockSpec((D, Dn), lambda i: (0, 0)),
        ],
        out_specs=pl.BlockSpec((RB, Dn), lambda i: (i, 0)),
        out_shape=jax.ShapeDtypeStruct((N, Dn), jnp.bfloat16),
    )(P, b, dinv, W)


def _tc_mid_id(P, b, dinv):
    D = P.shape[2]

    def body(p_ref, b_ref, dinv_ref, m_ref):
        dinv = dinv_ref[...]
        h = jnp.maximum(dinv * (p_ref[0].astype(jnp.float32) + p_ref[1].astype(jnp.float32)) + b_ref[...], 0.0)
        m_ref[...] = (h * dinv).astype(jnp.bfloat16)

    return pl.pallas_call(
        body,
        grid=(N // RB,),
        in_specs=[
            pl.BlockSpec((NC, RB, D), lambda i: (0, i, 0)),
            pl.BlockSpec((1, D), lambda i: (0, 0)),
            pl.BlockSpec((RB, 1), lambda i: (i, 0)),
        ],
        out_specs=pl.BlockSpec((RB, D), lambda i: (i, 0)),
        out_shape=jax.ShapeDtypeStruct((N, D), jnp.bfloat16),
    )(P, b, dinv)


def _tc_post(P, W3, b3, dinv, W1n):
    D = P.shape[2]

    def body(p_ref, w3_ref, b3_ref, dinv_ref, *rest):
        dinv = dinv_ref[...]
        g = dinv * (p_ref[0].astype(jnp.float32) + p_ref[1].astype(jnp.float32))
        t = g @ w3_ref[...] + b3_ref[...]
        h = jax.nn.softmax(t, axis=-1)
        if W1n is None:
            rest[-1][...] = h
        else:
            w1_ref = rest[0]
            rest[-1][...] = ((h @ w1_ref[...]) * dinv).astype(jnp.bfloat16)

    in_specs = [
        pl.BlockSpec((NC, RB, D), lambda i: (0, i, 0)),
        pl.BlockSpec((D, NF), lambda i: (0, 0)),
        pl.BlockSpec((1, NF), lambda i: (0, 0)),
        pl.BlockSpec((RB, 1), lambda i: (i, 0)),
    ]
    args = [P, W3, b3, dinv]
    if W1n is None:
        out_dim = NF
    else:
        out_dim = D1
        in_specs.append(pl.BlockSpec((NF, D1), lambda i: (0, 0)))
        args.append(W1n)
    return pl.pallas_call(
        body,
        grid=(N // RB,),
        in_specs=in_specs,
        out_specs=pl.BlockSpec((RB, out_dim), lambda i: (i, 0)),
        out_shape=jax.ShapeDtypeStruct(
            (N, out_dim), jnp.float32 if W1n is None else jnp.bfloat16),
    )(*args)


def kernel(x, edge_index,
           W11, b11, W21, b21, W31, b31,
           W12, b12, W22, b22, W32, b32,
           W13, b13, W23, b23, W33, b33,
           W14, b14, W24, b24, W34, b34,
           W15, b15, W25, b25, W35, b35):
    params = {
        1: (W11, b11, W21, b21, W31, b31),
        2: (W12, b12, W22, b22, W32, b32),
        3: (W13, b13, W23, b23, W33, b33),
        4: (W14, b14, W24, b24, W34, b34),
        5: (W15, b15, W25, b25, W35, b35),
    }
    loop = jnp.arange(N, dtype=jnp.int32)
    src = jnp.concatenate([edge_index[0], loop])
    dst = jnp.concatenate([edge_index[1], loop])
    pad = EPAD - src.shape[0]
    src = jnp.concatenate([src, jnp.zeros((pad,), jnp.int32)])
    dst = jnp.concatenate([dst, jnp.full((pad,), N, jnp.int32)])
    srcA = src.reshape(NC, NS, NCH, CH)
    dstA = dst.reshape(NC, NS, NCH, CH)

    degP = _make_deg()(dstA)
    dinv, M = _tc_prep(degP, x, W11)

    agg64 = _make_agg(D1)
    agg32 = _make_agg(D2)
    h = None
    for blk in range(1, 6):
        w1, b1, w2, b2, w3, b3 = params[blk]
        P = agg64(M, srcA, dstA)
        M = _tc_mid(P, b1.reshape(1, -1), dinv, w2)
        P = agg32(M, srcA, dstA)
        M = _tc_mid_id(P, b2.reshape(1, -1), dinv)
        P = agg32(M, srcA, dstA)
        if blk < 5:
            M = _tc_post(P, w3, b3.reshape(1, -1), dinv, params[blk + 1][0])
        else:
            h = _tc_post(P, w3, b3.reshape(1, -1), dinv, None)
    return h


# speedup vs baseline: 2.2867x; 1.9406x over previous
import functools

import jax
import jax.numpy as jnp
from jax import lax
from jax.experimental import pallas as pl
from jax.experimental.pallas import tpu as pltpu
from jax.experimental.pallas import tpu_sc as plsc

N = 10000
NF = 128
D1 = 64
D2 = 32
NC, NS = 2, 16
CH = 128
NCH = 82
EPW = NCH * CH
EPAD = NC * NS * EPW
NP = 10240
RPS = NP // NS
RB = 1000


def _sc_mesh():
    return plsc.VectorSubcoreMesh(
        core_axis_name="c", subcore_axis_name="s", num_cores=NC, num_subcores=NS
    )


@functools.lru_cache(maxsize=None)
def _make_agg(D):
    @functools.partial(
        pl.kernel,
        out_type=jax.ShapeDtypeStruct((NC, NP, D), jnp.bfloat16),
        mesh=_sc_mesh(),
        scratch_types=[
            pltpu.VMEM((NCH, CH), jnp.int32),
            pltpu.VMEM((NCH, CH), jnp.int32),
            pltpu.VMEM((CH, D), jnp.bfloat16),
            pltpu.VMEM_SHARED((NP, D), jnp.bfloat16),
            pltpu.VMEM_SHARED((N, D), jnp.bfloat16),
            pltpu.SemaphoreType.DMA,
        ],
        compiler_params=pltpu.CompilerParams(use_tc_tiling_on_sc=False),
    )
    def agg(m_hbm, src_hbm, dst_hbm, out_hbm, src_v, dst_v, rows_v, acc_sh,
            m_sh, sem):
        c = lax.axis_index("c")
        s = lax.axis_index("s")
        pltpu.sync_copy(src_hbm.at[c, s], src_v)
        pltpu.sync_copy(dst_hbm.at[c, s], dst_v)
        pltpu.sync_copy(m_hbm.at[pl.ds(s * (N // NS), N // NS)],
                        m_sh.at[pl.ds(s * (N // NS), N // NS)])

        zero32 = jnp.zeros((32,), jnp.bfloat16)

        def zrow(i, carry):
            for j in range(D // 32):
                rows_v[i, pl.ds(j * 32, 32)] = zero32
            return carry

        lax.fori_loop(0, CH, zrow, 0)
        base = s * RPS
        for k in range(RPS // CH):
            pltpu.sync_copy(rows_v, acc_sh.at[pl.ds(base + k * CH, CH)])
        plsc.subcore_barrier()

        def body(j, carry):
            pltpu.async_copy(m_sh.at[src_v.at[j]], rows_v, sem).wait()
            pltpu.sync_copy(rows_v, acc_sh.at[dst_v.at[j]], add=True)
            return carry

        lax.fori_loop(0, NCH, body, 0)
        plsc.subcore_barrier()
        pltpu.sync_copy(acc_sh.at[pl.ds(base, RPS)], out_hbm.at[c, pl.ds(base, RPS)])

    return agg


@functools.lru_cache(maxsize=None)
def _make_deg():
    @functools.partial(
        pl.kernel,
        out_type=jax.ShapeDtypeStruct((NC, NP, 16), jnp.float32),
        mesh=_sc_mesh(),
        scratch_types=[
            pltpu.VMEM((NCH, CH), jnp.int32),
            pltpu.VMEM((CH, 16), jnp.float32),
            pltpu.VMEM_SHARED((NP, 16), jnp.float32),
        ],
        compiler_params=pltpu.CompilerParams(use_tc_tiling_on_sc=False),
    )
    def deg(dst_hbm, out_hbm, dst_v, rows_v, acc_sh):
        c = lax.axis_index("c")
        s = lax.axis_index("s")
        pltpu.sync_copy(dst_hbm.at[c, s], dst_v)

        zero16 = jnp.zeros((16,), jnp.float32)

        def zrow(i, carry):
            rows_v[i, pl.ds(0, 16)] = zero16
            return carry

        lax.fori_loop(0, CH, zrow, 0)
        base = s * RPS
        for k in range(RPS // CH):
            pltpu.sync_copy(rows_v, acc_sh.at[pl.ds(base + k * CH, CH)])
        plsc.subcore_barrier()

        one16 = jnp.ones((16,), jnp.float32)

        def orow(i, carry):
            rows_v[i, pl.ds(0, 16)] = one16
            return carry

        lax.fori_loop(0, CH, orow, 0)

        def body(j, carry):
            pltpu.sync_copy(rows_v, acc_sh.at[dst_v.at[j]], add=True)
            return carry

        lax.fori_loop(0, NCH, body, 0)
        plsc.subcore_barrier()
        pltpu.sync_copy(acc_sh.at[pl.ds(base, RPS)], out_hbm.at[c, pl.ds(base, RPS)])

    return deg


def _tc_prep(degP, x, W1):
    def body(degp, x_ref, w_ref, dinv_ref, m_ref):
        d = degp[0, :, 0:1] + degp[1, :, 0:1]
        dinv = lax.rsqrt(d)
        dinv_ref[...] = dinv
        m_ref[...] = ((x_ref[...] @ w_ref[...]) * dinv).astype(jnp.bfloat16)

    return pl.pallas_call(
        body,
        grid=(N // RB,),
        in_specs=[
            pl.BlockSpec((NC, RB, 16), lambda i: (0, i, 0)),
            pl.BlockSpec((RB, NF), lambda i: (i, 0)),
            pl.BlockSpec((NF, D1), lambda i: (0, 0)),
        ],
        out_specs=[
            pl.BlockSpec((RB, 1), lambda i: (i, 0)),
            pl.BlockSpec((RB, D1), lambda i: (i, 0)),
        ],
        out_shape=[
            jax.ShapeDtypeStruct((N, 1), jnp.float32),
            jax.ShapeDtypeStruct((N, D1), jnp.bfloat16),
        ],
    )(degP, x, W1)


def _tc_mid(P, b, dinv, W):
    D = P.shape[2]
    Dn = W.shape[1]

    def body(p_ref, b_ref, dinv_ref, w_ref, m_ref):
        dinv = dinv_ref[...]
        h = jnp.maximum(dinv * (p_ref[0].astype(jnp.float32) + p_ref[1].astype(jnp.float32)) + b_ref[...], 0.0)
        m_ref[...] = ((h @ w_ref[...]) * dinv).astype(jnp.bfloat16)

    return pl.pallas_call(
        body,
        grid=(N // RB,),
        in_specs=[
            pl.BlockSpec((NC, RB, D), lambda i: (0, i, 0)),
            pl.BlockSpec((1, D), lambda i: (0, 0)),
            pl.BlockSpec((RB, 1), lambda i: (i, 0)),
            pl.BlockSpec((D, Dn), lambda i: (0, 0)),
        ],
        out_specs=pl.BlockSpec((RB, Dn), lambda i: (i, 0)),
        out_shape=jax.ShapeDtypeStruct((N, Dn), jnp.bfloat16),
    )(P, b, dinv, W)


def _tc_mid_id(P, b, dinv):
    D = P.shape[2]

    def body(p_ref, b_ref, dinv_ref, m_ref):
        dinv = dinv_ref[...]
        h = jnp.maximum(dinv * (p_ref[0].astype(jnp.float32) + p_ref[1].astype(jnp.float32)) + b_ref[...], 0.0)
        m_ref[...] = (h * dinv).astype(jnp.bfloat16)

    return pl.pallas_call(
        body,
        grid=(N // RB,),
        in_specs=[
            pl.BlockSpec((NC, RB, D), lambda i: (0, i, 0)),
            pl.BlockSpec((1, D), lambda i: (0, 0)),
            pl.BlockSpec((RB, 1), lambda i: (i, 0)),
        ],
        out_specs=pl.BlockSpec((RB, D), lambda i: (i, 0)),
        out_shape=jax.ShapeDtypeStruct((N, D), jnp.bfloat16),
    )(P, b, dinv)


def _tc_post(P, W3, b3, dinv, W1n):
    D = P.shape[2]

    def body(p_ref, w3_ref, b3_ref, dinv_ref, *rest):
        dinv = dinv_ref[...]
        g = dinv * (p_ref[0].astype(jnp.float32) + p_ref[1].astype(jnp.float32))
        t = g @ w3_ref[...] + b3_ref[...]
        h = jax.nn.softmax(t, axis=-1)
        if W1n is None:
            rest[-1][...] = h
        else:
            w1_ref = rest[0]
            rest[-1][...] = ((h @ w1_ref[...]) * dinv).astype(jnp.bfloat16)

    in_specs = [
        pl.BlockSpec((NC, RB, D), lambda i: (0, i, 0)),
        pl.BlockSpec((D, NF), lambda i: (0, 0)),
        pl.BlockSpec((1, NF), lambda i: (0, 0)),
        pl.BlockSpec((RB, 1), lambda i: (i, 0)),
    ]
    args = [P, W3, b3, dinv]
    if W1n is None:
        out_dim = NF
    else:
        out_dim = D1
        in_specs.append(pl.BlockSpec((NF, D1), lambda i: (0, 0)))
        args.append(W1n)
    return pl.pallas_call(
        body,
        grid=(N // RB,),
        in_specs=in_specs,
        out_specs=pl.BlockSpec((RB, out_dim), lambda i: (i, 0)),
        out_shape=jax.ShapeDtypeStruct(
            (N, out_dim), jnp.float32 if W1n is None else jnp.bfloat16),
    )(*args)


def kernel(x, edge_index,
           W11, b11, W21, b21, W31, b31,
           W12, b12, W22, b22, W32, b32,
           W13, b13, W23, b23, W33, b33,
           W14, b14, W24, b24, W34, b34,
           W15, b15, W25, b25, W35, b35):
    params = {
        1: (W11, b11, W21, b21, W31, b31),
        2: (W12, b12, W22, b22, W32, b32),
        3: (W13, b13, W23, b23, W33, b33),
        4: (W14, b14, W24, b24, W34, b34),
        5: (W15, b15, W25, b25, W35, b35),
    }
    loop = jnp.arange(N, dtype=jnp.int32)
    src = jnp.concatenate([edge_index[0], loop])
    dst = jnp.concatenate([edge_index[1], loop])
    pad = EPAD - src.shape[0]
    src = jnp.concatenate([src, jnp.zeros((pad,), jnp.int32)])
    dst = jnp.concatenate([dst, jnp.full((pad,), N, jnp.int32)])
    srcA = src.reshape(NC, NS, NCH, CH)
    dstA = dst.reshape(NC, NS, NCH, CH)

    degP = _make_deg()(dstA)
    dinv, M = _tc_prep(degP, x, W11)

    agg64 = _make_agg(D1)
    agg32 = _make_agg(D2)
    h = None
    for blk in range(1, 6):
        w1, b1, w2, b2, w3, b3 = params[blk]
        P = agg64(M, srcA, dstA)
        M = _tc_mid(P, b1.reshape(1, -1), dinv, w2)
        P = agg32(M, srcA, dstA)
        M = _tc_mid_id(P, b2.reshape(1, -1), dinv)
        P = agg32(M, srcA, dstA)
        if blk < 5:
            M = _tc_post(P, w3, b3.reshape(1, -1), dinv, params[blk + 1][0])
        else:
            h = _tc_post(P, w3, b3.reshape(1, -1), dinv, None)
    return h


# speedup vs baseline: 2.3479x; 1.0268x over previous
import functools

import jax
import jax.numpy as jnp
from jax import lax
from jax.experimental import pallas as pl
from jax.experimental.pallas import tpu as pltpu
from jax.experimental.pallas import tpu_sc as plsc

N = 10000
NF = 128
D1 = 64
D2 = 32
NC, NS = 2, 16
CH = 256
NCH = 41
EPW = NCH * CH
EPAD = NC * NS * EPW
NP = 10240
RPS = NP // NS
RB = 1000


def _sc_mesh():
    return plsc.VectorSubcoreMesh(
        core_axis_name="c", subcore_axis_name="s", num_cores=NC, num_subcores=NS
    )


@functools.lru_cache(maxsize=None)
def _make_agg(D):
    @functools.partial(
        pl.kernel,
        out_type=jax.ShapeDtypeStruct((NC, NP, D), jnp.bfloat16),
        mesh=_sc_mesh(),
        scratch_types=[
            pltpu.VMEM((NCH, CH), jnp.int32),
            pltpu.VMEM((NCH, CH), jnp.int32),
            pltpu.VMEM((CH, D), jnp.bfloat16),
            pltpu.VMEM_SHARED((NP, D), jnp.bfloat16),
            pltpu.VMEM_SHARED((N, D), jnp.bfloat16),
            pltpu.SemaphoreType.DMA,
        ],
        compiler_params=pltpu.CompilerParams(use_tc_tiling_on_sc=False),
    )
    def agg(m_hbm, src_hbm, dst_hbm, out_hbm, src_v, dst_v, rows_v, acc_sh,
            m_sh, sem):
        c = lax.axis_index("c")
        s = lax.axis_index("s")
        pltpu.sync_copy(src_hbm.at[c, s], src_v)
        pltpu.sync_copy(dst_hbm.at[c, s], dst_v)
        pltpu.sync_copy(m_hbm.at[pl.ds(s * (N // NS), N // NS)],
                        m_sh.at[pl.ds(s * (N // NS), N // NS)])

        zero32 = jnp.zeros((32,), jnp.bfloat16)

        def zrow(i, carry):
            for j in range(D // 32):
                rows_v[i, pl.ds(j * 32, 32)] = zero32
            return carry

        lax.fori_loop(0, CH, zrow, 0)
        base = s * RPS
        for k in range(RPS // 128):
            pltpu.sync_copy(rows_v.at[pl.ds(0, 128)],
                            acc_sh.at[pl.ds(base + k * 128, 128)])
        plsc.subcore_barrier()

        def body(j, carry):
            pltpu.async_copy(m_sh.at[src_v.at[j]], rows_v, sem).wait()
            pltpu.sync_copy(rows_v, acc_sh.at[dst_v.at[j]], add=True)
            return carry

        lax.fori_loop(0, NCH, body, 0)
        plsc.subcore_barrier()
        pltpu.sync_copy(acc_sh.at[pl.ds(base, RPS)], out_hbm.at[c, pl.ds(base, RPS)])

    return agg


@functools.lru_cache(maxsize=None)
def _make_deg():
    @functools.partial(
        pl.kernel,
        out_type=jax.ShapeDtypeStruct((NC, NP, 16), jnp.float32),
        mesh=_sc_mesh(),
        scratch_types=[
            pltpu.VMEM((NCH, CH), jnp.int32),
            pltpu.VMEM((CH, 16), jnp.float32),
            pltpu.VMEM_SHARED((NP, 16), jnp.float32),
        ],
        compiler_params=pltpu.CompilerParams(use_tc_tiling_on_sc=False),
    )
    def deg(dst_hbm, out_hbm, dst_v, rows_v, acc_sh):
        c = lax.axis_index("c")
        s = lax.axis_index("s")
        pltpu.sync_copy(dst_hbm.at[c, s], dst_v)

        zero16 = jnp.zeros((16,), jnp.float32)

        def zrow(i, carry):
            rows_v[i, pl.ds(0, 16)] = zero16
            return carry

        lax.fori_loop(0, CH, zrow, 0)
        base = s * RPS
        for k in range(RPS // 128):
            pltpu.sync_copy(rows_v.at[pl.ds(0, 128)],
                            acc_sh.at[pl.ds(base + k * 128, 128)])
        plsc.subcore_barrier()

        one16 = jnp.ones((16,), jnp.float32)

        def orow(i, carry):
            rows_v[i, pl.ds(0, 16)] = one16
            return carry

        lax.fori_loop(0, CH, orow, 0)

        def body(j, carry):
            pltpu.sync_copy(rows_v, acc_sh.at[dst_v.at[j]], add=True)
            return carry

        lax.fori_loop(0, NCH, body, 0)
        plsc.subcore_barrier()
        pltpu.sync_copy(acc_sh.at[pl.ds(base, RPS)], out_hbm.at[c, pl.ds(base, RPS)])

    return deg


def _tc_prep(degP, x, W1):
    def body(degp, x_ref, w_ref, dinv_ref, m_ref):
        d = degp[0, :, 0:1] + degp[1, :, 0:1]
        dinv = lax.rsqrt(d)
        dinv_ref[...] = dinv
        m_ref[...] = ((x_ref[...] @ w_ref[...]) * dinv).astype(jnp.bfloat16)

    return pl.pallas_call(
        body,
        grid=(N // RB,),
        in_specs=[
            pl.BlockSpec((NC, RB, 16), lambda i: (0, i, 0)),
            pl.BlockSpec((RB, NF), lambda i: (i, 0)),
            pl.BlockSpec((NF, D1), lambda i: (0, 0)),
        ],
        out_specs=[
            pl.BlockSpec((RB, 1), lambda i: (i, 0)),
            pl.BlockSpec((RB, D1), lambda i: (i, 0)),
        ],
        out_shape=[
            jax.ShapeDtypeStruct((N, 1), jnp.float32),
            jax.ShapeDtypeStruct((N, D1), jnp.bfloat16),
        ],
    )(degP, x, W1)


def _tc_mid(P, b, dinv, W):
    D = P.shape[2]
    Dn = W.shape[1]

    def body(p_ref, b_ref, dinv_ref, w_ref, m_ref):
        dinv = dinv_ref[...]
        h = jnp.maximum(dinv * (p_ref[0].astype(jnp.float32) + p_ref[1].astype(jnp.float32)) + b_ref[...], 0.0)
        m_ref[...] = ((h @ w_ref[...]) * dinv).astype(jnp.bfloat16)

    return pl.pallas_call(
        body,
        grid=(N // RB,),
        in_specs=[
            pl.BlockSpec((NC, RB, D), lambda i: (0, i, 0)),
            pl.BlockSpec((1, D), lambda i: (0, 0)),
            pl.BlockSpec((RB, 1), lambda i: (i, 0)),
            pl.BlockSpec((D, Dn), lambda i: (0, 0)),
        ],
        out_specs=pl.BlockSpec((RB, Dn), lambda i: (i, 0)),
        out_shape=jax.ShapeDtypeStruct((N, Dn), jnp.bfloat16),
    )(P, b, dinv, W)


def _tc_mid_id(P, b, dinv):
    D = P.shape[2]

    def body(p_ref, b_ref, dinv_ref, m_ref):
        dinv = dinv_ref[...]
        h = jnp.maximum(dinv * (p_ref[0].astype(jnp.float32) + p_ref[1].astype(jnp.float32)) + b_ref[...], 0.0)
        m_ref[...] = (h * dinv).astype(jnp.bfloat16)

    return pl.pallas_call(
        body,
        grid=(N // RB,),
        in_specs=[
            pl.BlockSpec((NC, RB, D), lambda i: (0, i, 0)),
            pl.BlockSpec((1, D), lambda i: (0, 0)),
            pl.BlockSpec((RB, 1), lambda i: (i, 0)),
        ],
        out_specs=pl.BlockSpec((RB, D), lambda i: (i, 0)),
        out_shape=jax.ShapeDtypeStruct((N, D), jnp.bfloat16),
    )(P, b, dinv)


def _tc_post(P, W3, b3, dinv, W1n):
    D = P.shape[2]

    def body(p_ref, w3_ref, b3_ref, dinv_ref, *rest):
        dinv = dinv_ref[...]
        g = dinv * (p_ref[0].astype(jnp.float32) + p_ref[1].astype(jnp.float32))
        t = g @ w3_ref[...] + b3_ref[...]
        h = jax.nn.softmax(t, axis=-1)
        if W1n is None:
            rest[-1][...] = h
        else:
            w1_ref = rest[0]
            rest[-1][...] = ((h @ w1_ref[...]) * dinv).astype(jnp.bfloat16)

    in_specs = [
        pl.BlockSpec((NC, RB, D), lambda i: (0, i, 0)),
        pl.BlockSpec((D, NF), lambda i: (0, 0)),
        pl.BlockSpec((1, NF), lambda i: (0, 0)),
        pl.BlockSpec((RB, 1), lambda i: (i, 0)),
    ]
    args = [P, W3, b3, dinv]
    if W1n is None:
        out_dim = NF
    else:
        out_dim = D1
        in_specs.append(pl.BlockSpec((NF, D1), lambda i: (0, 0)))
        args.append(W1n)
    return pl.pallas_call(
        body,
        grid=(N // RB,),
        in_specs=in_specs,
        out_specs=pl.BlockSpec((RB, out_dim), lambda i: (i, 0)),
        out_shape=jax.ShapeDtypeStruct(
            (N, out_dim), jnp.float32 if W1n is None else jnp.bfloat16),
    )(*args)


def kernel(x, edge_index,
           W11, b11, W21, b21, W31, b31,
           W12, b12, W22, b22, W32, b32,
           W13, b13, W23, b23, W33, b33,
           W14, b14, W24, b24, W34, b34,
           W15, b15, W25, b25, W35, b35):
    params = {
        1: (W11, b11, W21, b21, W31, b31),
        2: (W12, b12, W22, b22, W32, b32),
        3: (W13, b13, W23, b23, W33, b33),
        4: (W14, b14, W24, b24, W34, b34),
        5: (W15, b15, W25, b25, W35, b35),
    }
    loop = jnp.arange(N, dtype=jnp.int32)
    src = jnp.concatenate([edge_index[0], loop])
    dst = jnp.concatenate([edge_index[1], loop])
    pad = EPAD - src.shape[0]
    src = jnp.concatenate([src, jnp.zeros((pad,), jnp.int32)])
    dst = jnp.concatenate([dst, jnp.full((pad,), N, jnp.int32)])
    srcA = src.reshape(NC, NS, NCH, CH)
    dstA = dst.reshape(NC, NS, NCH, CH)

    degP = _make_deg()(dstA)
    dinv, M = _tc_prep(degP, x, W11)

    agg64 = _make_agg(D1)
    agg32 = _make_agg(D2)
    h = None
    for blk in range(1, 6):
        w1, b1, w2, b2, w3, b3 = params[blk]
        P = agg64(M, srcA, dstA)
        M = _tc_mid(P, b1.reshape(1, -1), dinv, w2)
        P = agg32(M, srcA, dstA)
        M = _tc_mid_id(P, b2.reshape(1, -1), dinv)
        P = agg32(M, srcA, dstA)
        if blk < 5:
            M = _tc_post(P, w3, b3.reshape(1, -1), dinv, params[blk + 1][0])
        else:
            h = _tc_post(P, w3, b3.reshape(1, -1), dinv, None)
    return h


# speedup vs baseline: 2.3612x; 1.0057x over previous
import functools

import jax
import jax.numpy as jnp
from jax import lax
from jax.experimental import pallas as pl
from jax.experimental.pallas import tpu as pltpu
from jax.experimental.pallas import tpu_sc as plsc

N = 10000
NF = 128
D1 = 64
D2 = 32
NC, NS = 2, 16
CH = 256
NCH = 40
EPW = NCH * CH
EPAD = NC * NS * EPW
NP = 10240
RPS = NP // NS
RB = 1000


def _sc_mesh():
    return plsc.VectorSubcoreMesh(
        core_axis_name="c", subcore_axis_name="s", num_cores=NC, num_subcores=NS
    )


@functools.lru_cache(maxsize=None)
def _make_agg(D):
    @functools.partial(
        pl.kernel,
        out_type=jax.ShapeDtypeStruct((NC, NP, D), jnp.bfloat16),
        mesh=_sc_mesh(),
        scratch_types=[
            pltpu.VMEM((NCH, CH), jnp.int32),
            pltpu.VMEM((NCH, CH), jnp.int32),
            pltpu.VMEM((CH, D), jnp.bfloat16),
            pltpu.VMEM_SHARED((NP, D), jnp.bfloat16),
            pltpu.VMEM_SHARED((N, D), jnp.bfloat16),
            pltpu.SemaphoreType.DMA,
        ],
        compiler_params=pltpu.CompilerParams(use_tc_tiling_on_sc=False),
    )
    def agg(m_hbm, src_hbm, dst_hbm, out_hbm, src_v, dst_v, rows_v, acc_sh,
            m_sh, sem):
        c = lax.axis_index("c")
        s = lax.axis_index("s")
        pltpu.sync_copy(src_hbm.at[c, s], src_v)
        pltpu.sync_copy(dst_hbm.at[c, s], dst_v)
        pltpu.sync_copy(m_hbm.at[pl.ds(s * (N // NS), N // NS)],
                        m_sh.at[pl.ds(s * (N // NS), N // NS)])

        zero32 = jnp.zeros((32,), jnp.bfloat16)

        def zrow(i, carry):
            for j in range(D // 32):
                rows_v[i, pl.ds(j * 32, 32)] = zero32
            return carry

        lax.fori_loop(0, CH, zrow, 0)
        base = s * RPS
        for k in range(RPS // 128):
            pltpu.sync_copy(rows_v.at[pl.ds(0, 128)],
                            acc_sh.at[pl.ds(base + k * 128, 128)])
        plsc.subcore_barrier()

        def body(j, carry):
            pltpu.async_copy(m_sh.at[src_v.at[j]], rows_v, sem).wait()
            pltpu.sync_copy(rows_v, acc_sh.at[dst_v.at[j]], add=True)
            return carry

        lax.fori_loop(0, NCH, body, 0)
        plsc.subcore_barrier()
        pltpu.sync_copy(acc_sh.at[pl.ds(base, RPS)], out_hbm.at[c, pl.ds(base, RPS)])

    return agg


@functools.lru_cache(maxsize=None)
def _make_deg():
    @functools.partial(
        pl.kernel,
        out_type=jax.ShapeDtypeStruct((NC, NP, 16), jnp.float32),
        mesh=_sc_mesh(),
        scratch_types=[
            pltpu.VMEM((NCH, CH), jnp.int32),
            pltpu.VMEM((CH, 16), jnp.float32),
            pltpu.VMEM_SHARED((NP, 16), jnp.float32),
        ],
        compiler_params=pltpu.CompilerParams(use_tc_tiling_on_sc=False),
    )
    def deg(dst_hbm, out_hbm, dst_v, rows_v, acc_sh):
        c = lax.axis_index("c")
        s = lax.axis_index("s")
        pltpu.sync_copy(dst_hbm.at[c, s], dst_v)

        zero16 = jnp.zeros((16,), jnp.float32)

        def zrow(i, carry):
            rows_v[i, pl.ds(0, 16)] = zero16
            return carry

        lax.fori_loop(0, CH, zrow, 0)
        base = s * RPS
        for k in range(RPS // 128):
            pltpu.sync_copy(rows_v.at[pl.ds(0, 128)],
                            acc_sh.at[pl.ds(base + k * 128, 128)])
        plsc.subcore_barrier()

        one16 = jnp.ones((16,), jnp.float32)

        def orow(i, carry):
            rows_v[i, pl.ds(0, 16)] = one16
            return carry

        lax.fori_loop(0, CH, orow, 0)

        def body(j, carry):
            pltpu.sync_copy(rows_v, acc_sh.at[dst_v.at[j]], add=True)
            return carry

        lax.fori_loop(0, NCH, body, 0)
        plsc.subcore_barrier()
        pltpu.sync_copy(acc_sh.at[pl.ds(base, RPS)], out_hbm.at[c, pl.ds(base, RPS)])

    return deg


def _tc_prep(degP, x, W1):
    def body(degp, x_ref, w_ref, dinv_ref, m_ref):
        d = degp[0, :, 0:1] + degp[1, :, 0:1] + 1.0
        dinv = lax.rsqrt(d)
        dinv_ref[...] = dinv
        m_ref[...] = ((x_ref[...] @ w_ref[...]) * dinv).astype(jnp.bfloat16)

    return pl.pallas_call(
        body,
        grid=(N // RB,),
        in_specs=[
            pl.BlockSpec((NC, RB, 16), lambda i: (0, i, 0)),
            pl.BlockSpec((RB, NF), lambda i: (i, 0)),
            pl.BlockSpec((NF, D1), lambda i: (0, 0)),
        ],
        out_specs=[
            pl.BlockSpec((RB, 1), lambda i: (i, 0)),
            pl.BlockSpec((RB, D1), lambda i: (i, 0)),
        ],
        out_shape=[
            jax.ShapeDtypeStruct((N, 1), jnp.float32),
            jax.ShapeDtypeStruct((N, D1), jnp.bfloat16),
        ],
    )(degP, x, W1)


def _tc_mid(P, M_in, b, dinv, W):
    D = P.shape[2]
    Dn = W.shape[1]

    def body(p_ref, m_in_ref, b_ref, dinv_ref, w_ref, m_ref):
        dinv = dinv_ref[...]
        psum = (p_ref[0].astype(jnp.float32) + p_ref[1].astype(jnp.float32)
                + m_in_ref[...].astype(jnp.float32))
        h = jnp.maximum(dinv * psum + b_ref[...], 0.0)
        m_ref[...] = ((h @ w_ref[...]) * dinv).astype(jnp.bfloat16)

    return pl.pallas_call(
        body,
        grid=(N // RB,),
        in_specs=[
            pl.BlockSpec((NC, RB, D), lambda i: (0, i, 0)),
            pl.BlockSpec((RB, D), lambda i: (i, 0)),
            pl.BlockSpec((1, D), lambda i: (0, 0)),
            pl.BlockSpec((RB, 1), lambda i: (i, 0)),
            pl.BlockSpec((D, Dn), lambda i: (0, 0)),
        ],
        out_specs=pl.BlockSpec((RB, Dn), lambda i: (i, 0)),
        out_shape=jax.ShapeDtypeStruct((N, Dn), jnp.bfloat16),
    )(P, M_in, b, dinv, W)


def _tc_mid_id(P, M_in, b, dinv):
    D = P.shape[2]

    def body(p_ref, m_in_ref, b_ref, dinv_ref, m_ref):
        dinv = dinv_ref[...]
        psum = (p_ref[0].astype(jnp.float32) + p_ref[1].astype(jnp.float32)
                + m_in_ref[...].astype(jnp.float32))
        h = jnp.maximum(dinv * psum + b_ref[...], 0.0)
        m_ref[...] = (h * dinv).astype(jnp.bfloat16)

    return pl.pallas_call(
        body,
        grid=(N // RB,),
        in_specs=[
            pl.BlockSpec((NC, RB, D), lambda i: (0, i, 0)),
            pl.BlockSpec((RB, D), lambda i: (i, 0)),
            pl.BlockSpec((1, D), lambda i: (0, 0)),
            pl.BlockSpec((RB, 1), lambda i: (i, 0)),
        ],
        out_specs=pl.BlockSpec((RB, D), lambda i: (i, 0)),
        out_shape=jax.ShapeDtypeStruct((N, D), jnp.bfloat16),
    )(P, M_in, b, dinv)


def _tc_post(P, M_in, W3, b3, dinv, W1n):
    D = P.shape[2]

    def body(p_ref, m_in_ref, w3_ref, b3_ref, dinv_ref, *rest):
        dinv = dinv_ref[...]
        psum = (p_ref[0].astype(jnp.float32) + p_ref[1].astype(jnp.float32)
                + m_in_ref[...].astype(jnp.float32))
        g = dinv * psum
        t = g @ w3_ref[...] + b3_ref[...]
        h = jax.nn.softmax(t, axis=-1)
        if W1n is None:
            rest[-1][...] = h
        else:
            w1_ref = rest[0]
            rest[-1][...] = ((h @ w1_ref[...]) * dinv).astype(jnp.bfloat16)

    in_specs = [
        pl.BlockSpec((NC, RB, D), lambda i: (0, i, 0)),
        pl.BlockSpec((RB, D), lambda i: (i, 0)),
        pl.BlockSpec((D, NF), lambda i: (0, 0)),
        pl.BlockSpec((1, NF), lambda i: (0, 0)),
        pl.BlockSpec((RB, 1), lambda i: (i, 0)),
    ]
    args = [P, M_in, W3, b3, dinv]
    if W1n is None:
        out_dim = NF
    else:
        out_dim = D1
        in_specs.append(pl.BlockSpec((NF, D1), lambda i: (0, 0)))
        args.append(W1n)
    return pl.pallas_call(
        body,
        grid=(N // RB,),
        in_specs=in_specs,
        out_specs=pl.BlockSpec((RB, out_dim), lambda i: (i, 0)),
        out_shape=jax.ShapeDtypeStruct(
            (N, out_dim), jnp.float32 if W1n is None else jnp.bfloat16),
    )(*args)


def kernel(x, edge_index,
           W11, b11, W21, b21, W31, b31,
           W12, b12, W22, b22, W32, b32,
           W13, b13, W23, b23, W33, b33,
           W14, b14, W24, b24, W34, b34,
           W15, b15, W25, b25, W35, b35):
    params = {
        1: (W11, b11, W21, b21, W31, b31),
        2: (W12, b12, W22, b22, W32, b32),
        3: (W13, b13, W23, b23, W33, b33),
        4: (W14, b14, W24, b24, W34, b34),
        5: (W15, b15, W25, b25, W35, b35),
    }
    src = edge_index[0]
    dst = edge_index[1]
    pad = EPAD - src.shape[0]
    src = jnp.concatenate([src, jnp.zeros((pad,), jnp.int32)])
    dst = jnp.concatenate([dst, jnp.full((pad,), N, jnp.int32)])
    srcA = src.reshape(NC, NS, NCH, CH)
    dstA = dst.reshape(NC, NS, NCH, CH)

    degP = _make_deg()(dstA)
    dinv, M = _tc_prep(degP, x, W11)

    agg64 = _make_agg(D1)
    agg32 = _make_agg(D2)
    h = None
    for blk in range(1, 6):
        w1, b1, w2, b2, w3, b3 = params[blk]
        P = agg64(M, srcA, dstA)
        M2 = _tc_mid(P, M, b1.reshape(1, -1), dinv, w2)
        P = agg32(M2, srcA, dstA)
        M3 = _tc_mid_id(P, M2, b2.reshape(1, -1), dinv)
        P = agg32(M3, srcA, dstA)
        if blk < 5:
            M = _tc_post(P, M3, w3, b3.reshape(1, -1), dinv, params[blk + 1][0])
        else:
            h = _tc_post(P, M3, w3, b3.reshape(1, -1), dinv, None)
    return h
